# Initial kernel scaffold; baseline (speedup 1.0000x reference)
import jax
import jax.numpy as jnp
from jax.experimental import pallas as pl


def kernel(embedding, target, fc, path_idx, path_codes, path_mask):
    raise NotImplementedError("write your pallas kernel here")


# speedup vs baseline: 1.3518x; 1.3518x over previous
import functools

import jax
import jax.numpy as jnp
from jax import lax
from jax.experimental import pallas as pl
from jax.experimental.pallas import tpu as pltpu
from jax.experimental.pallas import tpu_sc as plsc

_L = 16


def _log1p_series(t):
    s = t / (2.0 + t)
    s2 = s * s
    return 2.0 * s * (1.0 + s2 * (1.0 / 3.0 + s2 * (1.0 / 5.0 + s2 * (1.0 / 7.0))))


def _make_sc_kernel(N, H, V, D, NW):
    TPW = N // NW
    G = TPW // _L
    R = D * _L
    HV = H // _L

    mesh = plsc.VectorSubcoreMesh(core_axis_name="c", subcore_axis_name="s")
    info = plsc.get_sparse_core_info()
    NC = info.num_cores

    @functools.partial(
        pl.kernel,
        mesh=mesh,
        out_type=jax.ShapeDtypeStruct((NW, 2 * _L), jnp.float32),
        compiler_params=pltpu.CompilerParams(needs_layout_passes=False),
        scratch_types=[
            pltpu.VMEM((V * D,), jnp.int32),
            pltpu.VMEM((V * D,), jnp.float32),
            pltpu.VMEM((V * D,), jnp.float32),
            pltpu.VMEM((TPW,), jnp.int32),
            pltpu.VMEM((TPW, H), jnp.float32),
            pltpu.VMEM((R,), jnp.int32),
            pltpu.VMEM((R, H), jnp.float32),
            pltpu.VMEM((2 * _L,), jnp.float32),
            pltpu.SemaphoreType.DMA,
        ],
    )
    def sc_kernel(emb_hbm, tgt_hbm, fc_hbm, pidx_hbm, pcode_hbm, pmask_hbm,
                  out_hbm, pidx_v, pcode_v, pmask_v, tgt_v, emb_v, idx_v,
                  w_v, acc_v, sem):
        wid = lax.axis_index("s") * NC + lax.axis_index("c")
        base = wid * TPW

        pltpu.sync_copy(pidx_hbm, pidx_v)
        pltpu.sync_copy(pcode_hbm, pcode_v)
        pltpu.sync_copy(pmask_hbm, pmask_v)
        pltpu.sync_copy(tgt_hbm.at[pl.ds(base, TPW)], tgt_v)
        pltpu.sync_copy(emb_hbm.at[pl.ds(base, TPW), :], emb_v)

        lane = lax.iota(jnp.int32, _L)
        d_clamp = jnp.minimum(lane, D - 1)
        d_valid = (lane < D).astype(jnp.float32)

        def group_body(g, carry):
            accb, accm = carry
            t16 = tgt_v[pl.ds(g * _L, _L)] * D
            for d in range(D):
                idx_v[pl.ds(d * _L, _L)] = plsc.load_gather(
                    pidx_v, [t16 + d])
            cp0 = pltpu.async_copy(
                fc_hbm.at[idx_v.at[pl.ds(0, 128)]], w_v.at[pl.ds(0, 128), :],
                sem)
            cp1 = pltpu.async_copy(
                fc_hbm.at[idx_v.at[pl.ds(128, R - 128)]],
                w_v.at[pl.ds(128, R - 128), :], sem)
            cp0.wait()
            cp1.wait()

            def token_body(k, kcarry):
                kaccb, kaccm = kcarry
                tok = g * _L + k
                e = [emb_v[tok, pl.ds(j * _L, _L)] for j in range(HV)]

                def depth_body(d, pred):
                    r = d * _L + k
                    part = w_v[r, pl.ds(0, _L)] * e[0]
                    for j in range(1, HV):
                        part = part + w_v[r, pl.ds(j * _L, _L)] * e[j]
                    tot = jnp.sum(part)
                    return jnp.where(lane == d, jnp.full((_L,), tot), pred)

                pred = lax.fori_loop(0, D, depth_body, jnp.zeros((_L,), jnp.float32))
                tsp = plsc.load_gather(tgt_v, [jnp.full((_L,), tok, jnp.int32)])
                fidx = tsp * D + d_clamp
                codes = plsc.load_gather(pcode_v, [fidx])
                msk = plsc.load_gather(pmask_v, [fidx]) * d_valid
                bce = (jnp.maximum(pred, 0.0) - pred * codes
                       + _log1p_series(jnp.exp(-jnp.abs(pred))))
                return kaccb + bce * msk, kaccm + msk

            return lax.fori_loop(0, _L, token_body, (accb, accm))

        zero = jnp.zeros((_L,), jnp.float32)
        accb, accm = lax.fori_loop(0, G, group_body, (zero, zero))
        acc_v[pl.ds(0, _L)] = accb
        acc_v[pl.ds(_L, _L)] = accm
        pltpu.sync_copy(acc_v, out_hbm.at[wid])

    return sc_kernel


@jax.jit
def kernel(embedding, target, fc, path_idx, path_codes, path_mask):
    H = embedding.shape[-1]
    emb = embedding.reshape(-1, H)
    t = target.reshape(-1).astype(jnp.int32)
    N = emb.shape[0]
    V, D = path_idx.shape
    NW = 32
    sc = _make_sc_kernel(N, H, V, D, NW)
    parts = sc(emb, t, fc, path_idx.reshape(-1),
               path_codes.astype(jnp.float32).reshape(-1),
               path_mask.astype(jnp.float32).reshape(-1))
    bce_sum = jnp.sum(parts[:, :_L])
    mask_sum = jnp.sum(parts[:, _L:])
    return bce_sum / mask_sum
